# baseline (device time: 38153 ns/iter reference)
import jax
import jax.numpy as jnp
from jax import lax
from jax.experimental import pallas as pl
from jax.experimental.pallas import tpu as pltpu


def kernel(Q, K, V):
    b, s, h, d = Q.shape
    bh = b * h
    scale = d ** -0.5

    def to_heads(x):
        return jnp.reshape(jnp.transpose(x, (0, 2, 1, 3)), (bh, s, d)).astype(
            jnp.bfloat16
        )

    Qb, Kb, Vb = to_heads(Q), to_heads(K), to_heads(V)

    def body(q_ref, k_ref, v_ref, o_ref, kg_ref, vg_ref, send_sems, recv_sems):
        my_x = lax.axis_index("x")
        my_y = lax.axis_index("y")
        my_z = lax.axis_index("z")
        partner = (my_x, my_y, 1 - my_z)

        bar = pltpu.get_barrier_semaphore()
        pl.semaphore_signal(
            bar, inc=1, device_id=partner, device_id_type=pl.DeviceIdType.MESH
        )
        pl.semaphore_wait(bar, 1)

        rk = pltpu.make_async_remote_copy(
            src_ref=k_ref,
            dst_ref=kg_ref,
            send_sem=send_sems.at[0],
            recv_sem=recv_sems.at[0],
            device_id=partner,
            device_id_type=pl.DeviceIdType.MESH,
        )
        rv = pltpu.make_async_remote_copy(
            src_ref=v_ref,
            dst_ref=vg_ref,
            send_sem=send_sems.at[1],
            recv_sem=recv_sems.at[1],
            device_id=partner,
            device_id_type=pl.DeviceIdType.MESH,
        )
        rk.start()
        rv.start()
        rk.wait()
        rv.wait()

        for i in range(bh):
            q = q_ref[i]
            s0 = lax.dot_general(
                q, k_ref[i], (((1,), (1,)), ((), ())),
                preferred_element_type=jnp.float32,
            ) * scale
            s1 = lax.dot_general(
                q, kg_ref[i], (((1,), (1,)), ((), ())),
                preferred_element_type=jnp.float32,
            ) * scale
            m = jnp.maximum(
                jnp.max(s0, axis=1, keepdims=True),
                jnp.max(s1, axis=1, keepdims=True),
            )
            p0 = jnp.exp(s0 - m)
            p1 = jnp.exp(s1 - m)
            l = jnp.sum(p0, axis=1, keepdims=True) + jnp.sum(
                p1, axis=1, keepdims=True
            )
            p0 = (p0 / l).astype(jnp.bfloat16)
            p1 = (p1 / l).astype(jnp.bfloat16)
            o = lax.dot_general(
                p0, v_ref[i], (((1,), (0,)), ((), ())),
                preferred_element_type=jnp.float32,
            ) + lax.dot_general(
                p1, vg_ref[i], (((1,), (0,)), ((), ())),
                preferred_element_type=jnp.float32,
            )
            o_ref[i] = o

    out = pl.pallas_call(
        body,
        out_shape=jax.ShapeDtypeStruct((bh, s, d), jnp.float32),
        in_specs=[
            pl.BlockSpec(memory_space=pltpu.VMEM),
            pl.BlockSpec(memory_space=pltpu.VMEM),
            pl.BlockSpec(memory_space=pltpu.VMEM),
        ],
        out_specs=pl.BlockSpec(memory_space=pltpu.VMEM),
        scratch_shapes=[
            pltpu.VMEM((bh, s, d), jnp.bfloat16),
            pltpu.VMEM((bh, s, d), jnp.bfloat16),
            pltpu.SemaphoreType.DMA((2,)),
            pltpu.SemaphoreType.DMA((2,)),
        ],
        compiler_params=pltpu.CompilerParams(collective_id=0),
    )(Qb, Kb, Vb)

    return jnp.transpose(jnp.reshape(out, (b, h, s, d)), (0, 2, 1, 3))


# device time: 32677 ns/iter; 1.1676x vs baseline; 1.1676x over previous
import jax
import jax.numpy as jnp
from jax import lax
from jax.experimental import pallas as pl
from jax.experimental.pallas import tpu as pltpu


def kernel(Q, K, V):
    b, s, h, d = Q.shape
    bh = b * h
    scale = d ** -0.5

    def to_heads(x):
        return jnp.reshape(jnp.transpose(x, (0, 2, 1, 3)), (bh, s, d)).astype(
            jnp.bfloat16
        )

    Qb = to_heads(Q * scale)
    Kb = to_heads(K)
    Vb = to_heads(V)

    def body(q_ref, k_ref, v_ref, o_ref, kg_ref, vg_ref, send_sems, recv_sems):
        my_x = lax.axis_index("x")
        my_y = lax.axis_index("y")
        my_z = lax.axis_index("z")
        partner = (my_x, my_y, 1 - my_z)

        bar = pltpu.get_barrier_semaphore()
        pl.semaphore_signal(
            bar, inc=1, device_id=partner, device_id_type=pl.DeviceIdType.MESH
        )
        pl.semaphore_wait(bar, 1)

        rk = pltpu.make_async_remote_copy(
            src_ref=k_ref,
            dst_ref=kg_ref,
            send_sem=send_sems.at[0],
            recv_sem=recv_sems.at[0],
            device_id=partner,
            device_id_type=pl.DeviceIdType.MESH,
        )
        rv = pltpu.make_async_remote_copy(
            src_ref=v_ref,
            dst_ref=vg_ref,
            send_sem=send_sems.at[1],
            recv_sem=recv_sems.at[1],
            device_id=partner,
            device_id_type=pl.DeviceIdType.MESH,
        )
        rk.start()
        rv.start()

        dn_qk = (((2,), (2,)), ((0,), (0,)))
        dn_pv = (((2,), (1,)), ((0,), (0,)))

        q = q_ref[...]
        s0 = lax.dot_general(
            q, k_ref[...], dn_qk, preferred_element_type=jnp.float32
        )
        p0 = jnp.exp(s0)
        l0 = jnp.sum(p0, axis=2)
        o0 = lax.dot_general(
            p0.astype(jnp.bfloat16),
            v_ref[...],
            dn_pv,
            preferred_element_type=jnp.float32,
        )

        rk.wait()
        s1 = lax.dot_general(
            q, kg_ref[...], dn_qk, preferred_element_type=jnp.float32
        )
        p1 = jnp.exp(s1)
        l1 = jnp.sum(p1, axis=2)
        rv.wait()
        o1 = lax.dot_general(
            p1.astype(jnp.bfloat16),
            vg_ref[...],
            dn_pv,
            preferred_element_type=jnp.float32,
        )

        inv_l = 1.0 / (l0 + l1)
        o_ref[...] = (o0 + o1) * inv_l[:, :, None]

    out = pl.pallas_call(
        body,
        out_shape=jax.ShapeDtypeStruct((bh, s, d), jnp.float32),
        in_specs=[
            pl.BlockSpec(memory_space=pltpu.VMEM),
            pl.BlockSpec(memory_space=pltpu.VMEM),
            pl.BlockSpec(memory_space=pltpu.VMEM),
        ],
        out_specs=pl.BlockSpec(memory_space=pltpu.VMEM),
        scratch_shapes=[
            pltpu.VMEM((bh, s, d), jnp.bfloat16),
            pltpu.VMEM((bh, s, d), jnp.bfloat16),
            pltpu.SemaphoreType.DMA((2,)),
            pltpu.SemaphoreType.DMA((2,)),
        ],
        compiler_params=pltpu.CompilerParams(collective_id=0),
    )(Qb, Kb, Vb)

    return jnp.transpose(jnp.reshape(out, (b, h, s, d)), (0, 2, 1, 3))
